# baseline (speedup 1.0000x reference)
import functools

import jax
import jax.numpy as jnp
from jax import lax
from jax.experimental import pallas as pl
from jax.experimental.pallas import tpu as pltpu
from jax.experimental.pallas import tpu_sc as plsc

_NC, _NS = 2, 16
_NW = _NC * _NS


def _sc_body(etop_hbm, ebot_hbm, out_hbm, top_v, bot_v, buf, sem):
    f2 = out_hbm.shape[1]
    bsz = out_hbm.shape[0]
    nch = f2 // _NW
    wid = lax.axis_index("s") * _NC + lax.axis_index("c")
    c0 = wid * nch

    @pl.when(c0 < f2 // 2)
    def _():
        pltpu.sync_copy(etop_hbm.at[pl.ds(c0, nch)], top_v)
        for cl in range(nch):
            vs = [top_v[cl, pl.ds(16 * j, 16)] for j in range(8)]

            def qq_body(qq, carry, cl=cl, vs=vs):
                for j in range(8):
                    buf[cl, qq, pl.ds(16 * j, 16)] = vs[j]
                return carry

            lax.fori_loop(0, 32, qq_body, 0)

    @pl.when(c0 >= f2 // 2)
    def _():
        pltpu.sync_copy(ebot_hbm.at[pl.ds(c0 - f2 // 2, nch)], bot_v)
        for cl in range(nch):

            def qq_body(qq, carry, cl=cl):
                for j in range(2):
                    v = bot_v[cl, pl.ds((2 * qq + j) * 16, 16)]
                    for k in range(4):
                        buf[cl, qq, pl.ds(j * 64 + k * 16, 16)] = v
                return carry

            lax.fori_loop(0, 32, qq_body, 0)

    copies = [
        pltpu.async_copy(buf, out_hbm.at[b, pl.ds(c0, nch)], sem)
        for b in range(bsz)
    ]
    for cp in copies:
        cp.wait()


def kernel(x, row_embed, col_embed):
    bsz, _, h, w = x.shape
    f = row_embed.shape[1]
    nch = 2 * f // _NW
    ct = col_embed[:w, :].T
    rt = row_embed[:h, :].T
    etop = jnp.concatenate([ct, ct], axis=1)
    ebot = jnp.broadcast_to(rt[:, :, None], (f, h, 16)).reshape(f, h * 16)

    mesh = plsc.VectorSubcoreMesh(
        core_axis_name="c", subcore_axis_name="s",
        num_cores=_NC, num_subcores=_NS,
    )
    run = functools.partial(
        pl.kernel,
        out_type=jax.ShapeDtypeStruct((bsz, 2 * f, h * w // 128, 128), jnp.float32),
        mesh=mesh,
        scratch_types=[
            pltpu.VMEM((nch, 128), jnp.float32),
            pltpu.VMEM((nch, h * 16), jnp.float32),
            pltpu.VMEM((nch, h * w // 128, 128), jnp.float32),
            pltpu.SemaphoreType.DMA,
        ],
    )(_sc_body)
    out = run(etop, ebot)
    return out.reshape(bsz, 2 * f, h, w)
